# Initial kernel scaffold; baseline (speedup 1.0000x reference)
import jax
import jax.numpy as jnp
from jax.experimental import pallas as pl


def kernel(cos_sin, hidden_states, Wqkv, Wo, Wg, Wgu, Wd):
    raise NotImplementedError("write your pallas kernel here")


# speedup vs baseline: 1.1540x; 1.1540x over previous
import jax
import jax.numpy as jnp
from jax.experimental import pallas as pl
from jax.experimental.pallas import tpu as pltpu

B, S, H = 2, 2048, 1024
NH, HD = 16, 64
E, K = 8, 2
INTER = 2816
EPS = 1e-05
BS = B * S
NA = BS * K
T = 256
NT = (NA + E * (T - 1) + T - 1) // T
PAD = NT * T
NC = 2
IC = INTER // NC


def _rms(x):
    var = jnp.mean(x * x, axis=-1, keepdims=True)
    return x * jax.lax.rsqrt(var + EPS)


def _attn_kernel(x_ref, cos_ref, sin_ref, wq_ref, wk_ref, wv_ref, o_ref):
    x = x_ref[0]
    q = jnp.dot(x, wq_ref[0], preferred_element_type=jnp.float32)
    k = jnp.dot(x, wk_ref[0], preferred_element_type=jnp.float32)
    v = jnp.dot(x, wv_ref[0], preferred_element_type=jnp.float32)
    c = cos_ref[...]
    s = sin_ref[...]

    def rope(t):
        t1 = t[:, : HD // 2]
        t2 = t[:, HD // 2:]
        return t * c + jnp.concatenate([-t2, t1], axis=-1) * s

    q = rope(q)
    k = rope(k)
    scores = jax.lax.dot_general(q, k, (((1,), (1,)), ((), ())),
                                 preferred_element_type=jnp.float32)
    scores = scores * 0.125
    probs = jax.nn.softmax(scores, axis=-1)
    o_ref[0, 0] = jnp.dot(probs, v, preferred_element_type=jnp.float32)


def _post_kernel(x_ref, a_ref, wo_ref, wg_ref, hs_ref, lg_ref):
    o = jnp.dot(a_ref[...], wo_ref[...], preferred_element_type=jnp.float32)
    hs = _rms(x_ref[...] + o)
    hs_ref[...] = hs
    lg_ref[...] = jnp.dot(hs, wg_ref[...], preferred_element_type=jnp.float32)


def _expert_kernel(eid_ref, gate_ref, x_ref, wg_ref, wu_ref, wd_ref, y_ref):
    del eid_ref
    c = pl.program_id(1)
    x = x_ref[...]
    g = jnp.dot(x, wg_ref[0], preferred_element_type=jnp.float32)
    u = jnp.dot(x, wu_ref[0], preferred_element_type=jnp.float32)
    hh = (g * jax.nn.sigmoid(g)) * u
    y = jnp.dot(hh, wd_ref[0], preferred_element_type=jnp.float32)
    y = y * gate_ref[0]

    @pl.when(c == 0)
    def _():
        y_ref[...] = y

    @pl.when(c != 0)
    def _():
        y_ref[...] += y


def _final_kernel(hs_ref, m_ref, o_ref):
    o_ref[...] = _rms(hs_ref[...] + m_ref[...])


def kernel(cos_sin, hidden_states, Wqkv, Wo, Wg, Wgu, Wd):
    cos = cos_sin[0]
    sin = cos_sin[1]

    wqkv_h = Wqkv.reshape(H, 3 * NH, HD).transpose(1, 0, 2)
    attn = pl.pallas_call(
        _attn_kernel,
        grid=(B, NH),
        in_specs=[
            pl.BlockSpec((1, S, H), lambda b, h: (b, 0, 0)),
            pl.BlockSpec((S, HD), lambda b, h: (0, 0)),
            pl.BlockSpec((S, HD), lambda b, h: (0, 0)),
            pl.BlockSpec((1, H, HD), lambda b, h: (h, 0, 0)),
            pl.BlockSpec((1, H, HD), lambda b, h: (NH + h, 0, 0)),
            pl.BlockSpec((1, H, HD), lambda b, h: (2 * NH + h, 0, 0)),
        ],
        out_specs=pl.BlockSpec((1, 1, S, HD), lambda b, h: (b, h, 0, 0)),
        out_shape=jax.ShapeDtypeStruct((B, NH, S, HD), jnp.float32),
    )(hidden_states, cos, sin, wqkv_h, wqkv_h, wqkv_h)

    x_flat = hidden_states.reshape(BS, H)
    a_flat = attn.transpose(0, 2, 1, 3).reshape(BS, H)
    RB = 512
    hs, logits = pl.pallas_call(
        _post_kernel,
        grid=(BS // RB,),
        in_specs=[
            pl.BlockSpec((RB, H), lambda i: (i, 0)),
            pl.BlockSpec((RB, H), lambda i: (i, 0)),
            pl.BlockSpec((H, H), lambda i: (0, 0)),
            pl.BlockSpec((H, E), lambda i: (0, 0)),
        ],
        out_specs=[
            pl.BlockSpec((RB, H), lambda i: (i, 0)),
            pl.BlockSpec((RB, E), lambda i: (i, 0)),
        ],
        out_shape=[
            jax.ShapeDtypeStruct((BS, H), jnp.float32),
            jax.ShapeDtypeStruct((BS, E), jnp.float32),
        ],
    )(x_flat, a_flat, Wo, Wg)

    probs = jax.nn.softmax(logits, axis=-1)
    topv, topi = jax.lax.top_k(probs, K)
    topw = topv / jnp.clip(jnp.sum(topv, axis=-1, keepdims=True), 1e-08, None)
    eflat = topi.reshape(-1).astype(jnp.int32)
    wflat = topw.reshape(-1)
    order = jnp.argsort(eflat, stable=True).astype(jnp.int32)
    gsz = jnp.zeros((E,), jnp.int32).at[eflat].add(1)
    psz = ((gsz + T - 1) // T) * T
    gstart = jnp.concatenate([jnp.zeros((1,), jnp.int32),
                              jnp.cumsum(gsz)[:-1]])
    pcum = jnp.cumsum(psz)
    pstart = jnp.concatenate([jnp.zeros((1,), jnp.int32), pcum[:-1]])
    slots = jnp.arange(PAD, dtype=jnp.int32)
    e_of = jnp.minimum(jnp.searchsorted(pcum, slots, side='right'),
                       E - 1).astype(jnp.int32)
    j = slots - pstart[e_of]
    valid = j < gsz[e_of]
    src = jnp.minimum(gstart[e_of] + jnp.minimum(j, jnp.maximum(gsz[e_of] - 1, 0)),
                      NA - 1)
    pair = order[src]
    row = jnp.where(valid, pair // K, 0)
    gate = jnp.where(valid, wflat[pair], 0.0)
    tile_e = e_of[::T]

    xs = hs[row]
    gate3 = gate.reshape(NT, T, 1)

    ys = pl.pallas_call(
        _expert_kernel,
        grid_spec=pltpu.PrefetchScalarGridSpec(
            num_scalar_prefetch=1,
            grid=(NT, NC),
            in_specs=[
                pl.BlockSpec((1, T, 1), lambda t, c, eid: (t, 0, 0)),
                pl.BlockSpec((T, H), lambda t, c, eid: (t, 0)),
                pl.BlockSpec((1, H, IC), lambda t, c, eid: (eid[t], 0, c)),
                pl.BlockSpec((1, H, IC), lambda t, c, eid: (eid[t], 0, NC + c)),
                pl.BlockSpec((1, IC, H), lambda t, c, eid: (eid[t], c, 0)),
            ],
            out_specs=pl.BlockSpec((T, H), lambda t, c, eid: (t, 0)),
        ),
        out_shape=jax.ShapeDtypeStruct((PAD, H), jnp.float32),
    )(tile_e, gate3, xs, Wgu, Wgu, Wd)

    mixed = jnp.zeros((BS, H), jnp.float32).at[row].add(ys)

    out = pl.pallas_call(
        _final_kernel,
        grid=(BS // RB,),
        in_specs=[
            pl.BlockSpec((RB, H), lambda i: (i, 0)),
            pl.BlockSpec((RB, H), lambda i: (i, 0)),
        ],
        out_specs=pl.BlockSpec((RB, H), lambda i: (i, 0)),
        out_shape=jax.ShapeDtypeStruct((BS, H), jnp.float32),
    )(hs, mixed)

    return out.reshape(B, S, H)
